# baseline (speedup 1.0000x reference)
import jax
import jax.numpy as jnp
from jax.experimental import pallas as pl
from jax.experimental.pallas import tpu as pltpu

_N = 9216
_D = 64
_M = 1024
_BLK = 1152


def _rowsum64(y):
    s = y[:, 0:8]
    for k in range(1, 8):
        s = s + y[:, 8 * k:8 * (k + 1)]
    s = s[:, 0:4] + s[:, 4:8]
    s = s[:, 0:2] + s[:, 2:4]
    return s[:, 0:1] + s[:, 1:2]


def _colsum64(y):
    s = y[0:8, :]
    for k in range(1, 8):
        s = s + y[8 * k:8 * (k + 1), :]
    s = s[0:4, :] + s[4:8, :]
    s = s[0:2, :] + s[2:4, :]
    return s[0:1, :] + s[1:2, :]


def _vq_kernel(x_ref, e_ref, et_ref, out_ref, stats_ref, cnt_ref, acc_ref):
    i = pl.program_id(0)
    nsteps = pl.num_programs(0)

    @pl.when(i == 0)
    def _init():
        cnt_ref[...] = jnp.zeros_like(cnt_ref)
        acc_ref[...] = jnp.zeros_like(acc_ref)

    x = x_ref[...]
    e = e_ref[...]
    et = et_ref[...]
    xsq = _rowsum64(x * x)
    esq = _colsum64(et * et)
    xe = jax.lax.dot_general(
        x, e, (((1,), (1,)), ((), ())),
        preferred_element_type=jnp.float32)
    d2 = (xsq + esq) - 2.0 * xe
    d2 = jnp.maximum(d2, 0.0)

    minval = jnp.min(d2, axis=1, keepdims=True)
    lanes = jax.lax.broadcasted_iota(jnp.int32, (_BLK, _M), 1)
    idx = jnp.min(jnp.where(d2 == minval, lanes, _M), axis=1, keepdims=True)
    onehot = (lanes == idx).astype(jnp.float32)

    quant = jax.lax.dot_general(
        onehot, e, (((1,), (0,)), ((), ())),
        preferred_element_type=jnp.float32,
        precision=jax.lax.Precision.HIGHEST)
    out_ref[...] = quant

    diff = x - quant
    acc_ref[...] += jnp.sum(diff * diff).reshape(1, 1)
    cnt_ref[...] += jnp.sum(onehot, axis=0, keepdims=True)

    @pl.when(i == nsteps - 1)
    def _finalize():
        avg = cnt_ref[...] / float(_N)
        perp = jnp.exp(-jnp.sum(avg * jnp.log(avg + 1e-10)))
        recon = acc_ref[0, 0] / float(_N * _D)
        row = jax.lax.broadcasted_iota(jnp.int32, (8, 128), 0)
        stats = jnp.where(row == 0, recon,
                          jnp.where(row == 1, 0.25 * recon,
                                    jnp.where(row == 2, perp, 0.0)))
        stats_ref[...] = stats


def kernel(input, embedding):
    x = input.reshape(_N, _D)
    grid = _N // _BLK
    out, stats = pl.pallas_call(
        _vq_kernel,
        grid=(grid,),
        in_specs=[
            pl.BlockSpec((_BLK, _D), lambda i: (i, 0)),
            pl.BlockSpec((_M, _D), lambda i: (0, 0)),
            pl.BlockSpec((_D, _M), lambda i: (0, 0)),
        ],
        out_specs=[
            pl.BlockSpec((_BLK, _D), lambda i: (i, 0)),
            pl.BlockSpec((8, 128), lambda i: (0, 0)),
        ],
        out_shape=[
            jax.ShapeDtypeStruct((_N, _D), jnp.float32),
            jax.ShapeDtypeStruct((8, 128), jnp.float32),
        ],
        scratch_shapes=[
            pltpu.VMEM((1, _M), jnp.float32),
            pltpu.VMEM((1, 1), jnp.float32),
        ],
    )(x, embedding, embedding.T)
    quantised_st = out.reshape(input.shape)
    codebook_loss = stats[0, 0]
    commitment_loss = stats[1, 0]
    perplexity = stats[2, 0]
    return (quantised_st, commitment_loss, codebook_loss, perplexity)


# speedup vs baseline: 1.4298x; 1.4298x over previous
import jax
import jax.numpy as jnp
from jax.experimental import pallas as pl
from jax.experimental.pallas import tpu as pltpu
from jax.experimental.pallas import tpu_sc as plsc

_N = 9216
_D = 64
_M = 1024
_BLK = 1152

_GATHER_WINDOW = 128


def _rowsum64(y):
    s = y[:, 0:8]
    for k in range(1, 8):
        s = s + y[:, 8 * k:8 * (k + 1)]
    s = s[:, 0:4] + s[:, 4:8]
    s = s[:, 0:2] + s[:, 2:4]
    return s[:, 0:1] + s[:, 1:2]


def _colsum64(y):
    s = y[0:8, :]
    for k in range(1, 8):
        s = s + y[8 * k:8 * (k + 1), :]
    s = s[0:4, :] + s[4:8, :]
    s = s[0:2, :] + s[2:4, :]
    return s[0:1, :] + s[1:2, :]


def _vq_kernel(x_ref, e_ref, et_ref, idx_ref, stats_ref, cnt_ref, acc_ref):
    i = pl.program_id(0)
    nsteps = pl.num_programs(0)

    @pl.when(i == 0)
    def _init():
        cnt_ref[...] = jnp.zeros_like(cnt_ref)
        acc_ref[...] = jnp.zeros_like(acc_ref)

    x = x_ref[...]
    e = e_ref[...]
    et = et_ref[...]
    xsq = _rowsum64(x * x)
    esq = _colsum64(et * et)
    xe = jax.lax.dot_general(
        x, e, (((1,), (1,)), ((), ())),
        preferred_element_type=jnp.float32)
    d2 = (xsq + esq) - 2.0 * xe
    d2 = jnp.maximum(d2, 0.0)

    minval = jnp.min(d2, axis=1, keepdims=True)
    lanes = jax.lax.broadcasted_iota(jnp.int32, (_BLK, _M), 1)
    idx = jnp.min(jnp.where(d2 == minval, lanes, _M), axis=1, keepdims=True)
    idx_ref[...] = idx

    onehot = (lanes == idx).astype(jnp.float32)
    acc_ref[...] += jnp.sum(minval).reshape(1, 1)
    cnt_ref[...] += jnp.sum(onehot, axis=0, keepdims=True)

    @pl.when(i == nsteps - 1)
    def _finalize():
        avg = cnt_ref[...] / float(_N)
        perp = jnp.exp(-jnp.sum(avg * jnp.log(avg + 1e-10)))
        recon = acc_ref[0, 0] / float(_N * _D)
        row = jax.lax.broadcasted_iota(jnp.int32, (8, 128), 0)
        stats = jnp.where(row == 0, recon,
                          jnp.where(row == 1, 0.25 * recon,
                                    jnp.where(row == 2, perp, 0.0)))
        stats_ref[...] = stats


def _sc_gather(e128, indices):
    mesh = plsc.VectorSubcoreMesh(core_axis_name="c", subcore_axis_name="s")

    @pl.kernel(out_type=jax.ShapeDtypeStruct((_N, 128), jnp.float32),
               mesh=mesh)
    def gather_kernel(e_hbm, i_hbm, o_hbm):
        def body(i_vmem, o_vmem):
            pltpu.sync_copy(e_hbm.at[i_vmem.at[0]], o_vmem)

        pltpu.emit_pipeline(
            body,
            grid=(_N // _GATHER_WINDOW,),
            in_specs=[pl.BlockSpec((1, _GATHER_WINDOW),
                                   index_map=lambda i: (0, i))],
            out_specs=[pl.BlockSpec((_GATHER_WINDOW, 128),
                                    index_map=lambda i: (i, 0))],
            core_axis_name=("c", "s"),
            dimension_semantics=(pltpu.PARALLEL,),
        )(i_hbm, o_hbm)

    return gather_kernel(e128, indices)


def kernel(input, embedding):
    x = input.reshape(_N, _D)
    grid = _N // _BLK
    idx_col, stats = pl.pallas_call(
        _vq_kernel,
        grid=(grid,),
        in_specs=[
            pl.BlockSpec((_BLK, _D), lambda i: (i, 0)),
            pl.BlockSpec((_M, _D), lambda i: (0, 0)),
            pl.BlockSpec((_D, _M), lambda i: (0, 0)),
        ],
        out_specs=[
            pl.BlockSpec((_BLK, 1), lambda i: (i, 0)),
            pl.BlockSpec((8, 128), lambda i: (0, 0)),
        ],
        out_shape=[
            jax.ShapeDtypeStruct((_N, 1), jnp.int32),
            jax.ShapeDtypeStruct((8, 128), jnp.float32),
        ],
        scratch_shapes=[
            pltpu.VMEM((1, _M), jnp.float32),
            pltpu.VMEM((1, 1), jnp.float32),
        ],
    )(x, embedding, embedding.T)
    e128 = jnp.pad(embedding, ((0, 0), (0, 128 - _D)))
    quant = _sc_gather(e128, idx_col.reshape(1, _N))
    quantised_st = quant[:, :_D].reshape(input.shape)
    codebook_loss = stats[0, 0]
    commitment_loss = stats[1, 0]
    perplexity = stats[2, 0]
    return (quantised_st, commitment_loss, codebook_loss, perplexity)


# speedup vs baseline: 1.6520x; 1.1554x over previous
import jax
import jax.numpy as jnp
from jax.experimental import pallas as pl
from jax.experimental.pallas import tpu as pltpu
from jax.experimental.pallas import tpu_sc as plsc

_N = 9216
_D = 64
_M = 1024
_BLK = 4608

_GATHER_WINDOW = 128


def _rowsum64(y):
    s = y[:, 0:8]
    for k in range(1, 8):
        s = s + y[:, 8 * k:8 * (k + 1)]
    s = s[:, 0:4] + s[:, 4:8]
    s = s[:, 0:2] + s[:, 2:4]
    return s[:, 0:1] + s[:, 1:2]


def _colsum64(y):
    s = y[0:8, :]
    for k in range(1, 8):
        s = s + y[8 * k:8 * (k + 1), :]
    s = s[0:4, :] + s[4:8, :]
    s = s[0:2, :] + s[2:4, :]
    return s[0:1, :] + s[1:2, :]


def _vq_kernel(x_ref, e_ref, et_ref, idx_ref, stats_ref, cnt_ref, acc_ref):
    i = pl.program_id(0)
    nsteps = pl.num_programs(0)

    @pl.when(i == 0)
    def _init():
        cnt_ref[...] = jnp.zeros_like(cnt_ref)
        acc_ref[...] = jnp.zeros_like(acc_ref)

    x = x_ref[...]
    e = e_ref[...]
    et = et_ref[...]
    xt = jnp.transpose(x)
    xsq = jnp.transpose(_colsum64(xt * xt))
    esq = _colsum64(et * et)
    xm = jax.lax.dot_general(
        -2.0 * x, e, (((1,), (1,)), ((), ())),
        preferred_element_type=jnp.float32)
    d2 = (xsq + esq) + xm
    d2 = jnp.maximum(d2, 0.0)

    minval = jnp.min(d2, axis=1, keepdims=True)
    lanes = jax.lax.broadcasted_iota(jnp.int32, (_BLK, _M), 1)
    idx = jnp.min(jnp.where(d2 == minval, lanes, _M), axis=1, keepdims=True)
    idx_ref[...] = idx

    onehot = (lanes == idx).astype(jnp.float32)
    acc_ref[...] += jnp.sum(minval).reshape(1, 1)
    cnt_ref[...] += jnp.sum(onehot, axis=0, keepdims=True)

    @pl.when(i == nsteps - 1)
    def _finalize():
        avg = cnt_ref[...] / float(_N)
        perp = jnp.exp(-jnp.sum(avg * jnp.log(avg + 1e-10)))
        recon = acc_ref[0, 0] / float(_N * _D)
        row = jax.lax.broadcasted_iota(jnp.int32, (8, 128), 0)
        stats = jnp.where(row == 0, recon,
                          jnp.where(row == 1, 0.25 * recon,
                                    jnp.where(row == 2, perp, 0.0)))
        stats_ref[...] = stats


def _sc_gather(e128, indices):
    mesh = plsc.VectorSubcoreMesh(core_axis_name="c", subcore_axis_name="s")

    @pl.kernel(out_type=jax.ShapeDtypeStruct((_N, 128), jnp.float32),
               mesh=mesh)
    def gather_kernel(e_hbm, i_hbm, o_hbm):
        def body(i_vmem, o_vmem):
            pltpu.sync_copy(e_hbm.at[i_vmem.at[0]], o_vmem)

        pltpu.emit_pipeline(
            body,
            grid=(_N // _GATHER_WINDOW,),
            in_specs=[pl.BlockSpec((1, _GATHER_WINDOW),
                                   index_map=lambda i: (0, i))],
            out_specs=[pl.BlockSpec((_GATHER_WINDOW, 128),
                                    index_map=lambda i: (i, 0))],
            core_axis_name=("c", "s"),
            dimension_semantics=(pltpu.PARALLEL,),
        )(i_hbm, o_hbm)

    return gather_kernel(e128, indices)


def kernel(input, embedding):
    x = input.reshape(_N, _D)
    grid = _N // _BLK
    idx_col, stats = pl.pallas_call(
        _vq_kernel,
        grid=(grid,),
        in_specs=[
            pl.BlockSpec((_BLK, _D), lambda i: (i, 0)),
            pl.BlockSpec((_M, _D), lambda i: (0, 0)),
            pl.BlockSpec((_D, _M), lambda i: (0, 0)),
        ],
        out_specs=[
            pl.BlockSpec((_BLK, 1), lambda i: (i, 0)),
            pl.BlockSpec((8, 128), lambda i: (0, 0)),
        ],
        out_shape=[
            jax.ShapeDtypeStruct((_N, 1), jnp.int32),
            jax.ShapeDtypeStruct((8, 128), jnp.float32),
        ],
        scratch_shapes=[
            pltpu.VMEM((1, _M), jnp.float32),
            pltpu.VMEM((1, 1), jnp.float32),
        ],
    )(x, embedding, embedding.T)
    e128 = jnp.pad(embedding, ((0, 0), (0, 128 - _D)))
    quant = _sc_gather(e128, idx_col.reshape(1, _N))
    quantised_st = quant[:, :_D].reshape(input.shape)
    codebook_loss = stats[0, 0]
    commitment_loss = stats[1, 0]
    perplexity = stats[2, 0]
    return (quantised_st, commitment_loss, codebook_loss, perplexity)


# speedup vs baseline: 1.7619x; 1.0665x over previous
import jax
import jax.numpy as jnp
from jax.experimental import pallas as pl
from jax.experimental.pallas import tpu as pltpu
from jax.experimental.pallas import tpu_sc as plsc

_N = 9216
_D = 64
_M = 1024
_BLK = 4608

_GATHER_WINDOW = 128


def _rowsum64(y):
    s = y[:, 0:8]
    for k in range(1, 8):
        s = s + y[:, 8 * k:8 * (k + 1)]
    s = s[:, 0:4] + s[:, 4:8]
    s = s[:, 0:2] + s[:, 2:4]
    return s[:, 0:1] + s[:, 1:2]


def _colsum64(y):
    s = y[0:8, :]
    for k in range(1, 8):
        s = s + y[8 * k:8 * (k + 1), :]
    s = s[0:4, :] + s[4:8, :]
    s = s[0:2, :] + s[2:4, :]
    return s[0:1, :] + s[1:2, :]


def _vq_kernel(x_ref, e_ref, et_ref, idx_ref, stats_ref, cnt_ref, acc_ref):
    i = pl.program_id(0)
    nsteps = pl.num_programs(0)

    @pl.when(i == 0)
    def _init():
        cnt_ref[...] = jnp.zeros_like(cnt_ref)
        acc_ref[...] = jnp.zeros_like(acc_ref)

    x = x_ref[...]
    e = e_ref[...]
    et = et_ref[...]
    xt = jnp.transpose(x)
    xsq = jnp.transpose(_colsum64(xt * xt))
    esq = _colsum64(et * et)
    xm = jax.lax.dot_general(
        -2.0 * x, e, (((1,), (1,)), ((), ())),
        preferred_element_type=jnp.float32)
    d2 = (xsq + esq) + xm
    d2 = jnp.maximum(d2, 0.0)

    minval = jnp.min(d2, axis=1, keepdims=True)
    lanes = jax.lax.broadcasted_iota(jnp.int32, (_BLK, _M), 1)
    idx = jnp.min(jnp.where(d2 == minval, lanes, _M), axis=1, keepdims=True)
    idx_ref[...] = jnp.transpose(idx)

    onehot = (lanes == idx).astype(jnp.float32)
    acc_ref[...] += jnp.sum(minval).reshape(1, 1)
    cnt_ref[...] += jnp.sum(onehot, axis=0, keepdims=True)

    @pl.when(i == nsteps - 1)
    def _finalize():
        avg = cnt_ref[...] / float(_N)
        perp = jnp.exp(-jnp.sum(avg * jnp.log(avg + 1e-10)))
        recon = acc_ref[0, 0] / float(_N * _D)
        row = jax.lax.broadcasted_iota(jnp.int32, (8, 128), 0)
        stats = jnp.where(row == 0, recon,
                          jnp.where(row == 1, 0.25 * recon,
                                    jnp.where(row == 2, perp, 0.0)))
        stats_ref[...] = stats


def _sc_gather(e128, indices):
    mesh = plsc.VectorSubcoreMesh(core_axis_name="c", subcore_axis_name="s")

    @pl.kernel(out_type=jax.ShapeDtypeStruct((_N, 128), jnp.float32),
               mesh=mesh)
    def gather_kernel(e_hbm, i_hbm, o_hbm):
        def body(i_vmem, o_vmem):
            pltpu.sync_copy(e_hbm.at[i_vmem.at[0]], o_vmem)

        pltpu.emit_pipeline(
            body,
            grid=(_N // _GATHER_WINDOW,),
            in_specs=[pl.BlockSpec((1, _GATHER_WINDOW),
                                   index_map=lambda i: (0, i))],
            out_specs=[pl.BlockSpec((_GATHER_WINDOW, 128),
                                    index_map=lambda i: (i, 0))],
            core_axis_name=("c", "s"),
            dimension_semantics=(pltpu.PARALLEL,),
        )(i_hbm, o_hbm)

    return gather_kernel(e128, indices)


def kernel(input, embedding):
    x = input.reshape(_N, _D)
    grid = _N // _BLK
    idx_col, stats = pl.pallas_call(
        _vq_kernel,
        grid=(grid,),
        in_specs=[
            pl.BlockSpec((_BLK, _D), lambda i: (i, 0)),
            pl.BlockSpec((_M, _D), lambda i: (0, 0)),
            pl.BlockSpec((_D, _M), lambda i: (0, 0)),
        ],
        out_specs=[
            pl.BlockSpec((1, _BLK), lambda i: (0, i)),
            pl.BlockSpec((8, 128), lambda i: (0, 0)),
        ],
        out_shape=[
            jax.ShapeDtypeStruct((1, _N), jnp.int32),
            jax.ShapeDtypeStruct((8, 128), jnp.float32),
        ],
        scratch_shapes=[
            pltpu.VMEM((1, _M), jnp.float32),
            pltpu.VMEM((1, 1), jnp.float32),
        ],
    )(x, embedding, embedding.T)
    e128 = jnp.pad(embedding, ((0, 0), (0, 128 - _D)))
    quant = _sc_gather(e128, idx_col)
    quantised_st = quant[:, :_D].reshape(input.shape)
    codebook_loss = stats[0, 0]
    commitment_loss = stats[1, 0]
    perplexity = stats[2, 0]
    return (quantised_st, commitment_loss, codebook_loss, perplexity)


# speedup vs baseline: 1.7621x; 1.0001x over previous
import jax
import jax.numpy as jnp
from jax.experimental import pallas as pl
from jax.experimental.pallas import tpu as pltpu
from jax.experimental.pallas import tpu_sc as plsc

_N = 9216
_D = 64
_M = 1024
_BLK = 2304

_GATHER_WINDOW = 128


def _rowsum64(y):
    s = y[:, 0:8]
    for k in range(1, 8):
        s = s + y[:, 8 * k:8 * (k + 1)]
    s = s[:, 0:4] + s[:, 4:8]
    s = s[:, 0:2] + s[:, 2:4]
    return s[:, 0:1] + s[:, 1:2]


def _colsum64(y):
    s = y[0:8, :]
    for k in range(1, 8):
        s = s + y[8 * k:8 * (k + 1), :]
    s = s[0:4, :] + s[4:8, :]
    s = s[0:2, :] + s[2:4, :]
    return s[0:1, :] + s[1:2, :]


def _vq_kernel(x_ref, e_ref, et_ref, idx_ref, stats_ref, cnt_ref, acc_ref):
    i = pl.program_id(0)
    nsteps = pl.num_programs(0)

    @pl.when(i == 0)
    def _init():
        cnt_ref[...] = jnp.zeros_like(cnt_ref)
        acc_ref[...] = jnp.zeros_like(acc_ref)

    x = x_ref[...]
    e = e_ref[...]
    et = et_ref[...]
    xt = jnp.transpose(x)
    xsq = jnp.transpose(_colsum64(xt * xt))
    esq = _colsum64(et * et)
    xm = jax.lax.dot_general(
        -2.0 * x, e, (((1,), (1,)), ((), ())),
        preferred_element_type=jnp.float32)
    d2 = (xsq + esq) + xm
    d2 = jnp.maximum(d2, 0.0)

    minval = jnp.min(d2, axis=1, keepdims=True)
    lanes = jax.lax.broadcasted_iota(jnp.int32, (_BLK, _M), 1)
    idx = jnp.min(jnp.where(d2 == minval, lanes, _M), axis=1, keepdims=True)
    idx_ref[...] = jnp.transpose(idx)

    onehot = (lanes == idx).astype(jnp.float32)
    acc_ref[...] += jnp.sum(minval).reshape(1, 1)
    cnt_ref[...] += jnp.sum(onehot, axis=0, keepdims=True)

    @pl.when(i == nsteps - 1)
    def _finalize():
        avg = cnt_ref[...] / float(_N)
        perp = jnp.exp(-jnp.sum(avg * jnp.log(avg + 1e-10)))
        recon = acc_ref[0, 0] / float(_N * _D)
        row = jax.lax.broadcasted_iota(jnp.int32, (8, 128), 0)
        stats = jnp.where(row == 0, recon,
                          jnp.where(row == 1, 0.25 * recon,
                                    jnp.where(row == 2, perp, 0.0)))
        stats_ref[...] = stats


def _sc_gather(e128, indices):
    mesh = plsc.VectorSubcoreMesh(core_axis_name="c", subcore_axis_name="s")

    @pl.kernel(out_type=jax.ShapeDtypeStruct((_N, 128), jnp.float32),
               mesh=mesh)
    def gather_kernel(e_hbm, i_hbm, o_hbm):
        def body(i_vmem, o_vmem):
            pltpu.sync_copy(e_hbm.at[i_vmem.at[0]], o_vmem)

        pltpu.emit_pipeline(
            body,
            grid=(_N // _GATHER_WINDOW,),
            in_specs=[pl.BlockSpec((1, _GATHER_WINDOW),
                                   index_map=lambda i: (0, i))],
            out_specs=[pl.BlockSpec((_GATHER_WINDOW, 128),
                                    index_map=lambda i: (i, 0))],
            core_axis_name=("c", "s"),
            dimension_semantics=(pltpu.PARALLEL,),
        )(i_hbm, o_hbm)

    return gather_kernel(e128, indices)


def kernel(input, embedding):
    x = input.reshape(_N, _D)
    grid = _N // _BLK
    idx_col, stats = pl.pallas_call(
        _vq_kernel,
        grid=(grid,),
        in_specs=[
            pl.BlockSpec((_BLK, _D), lambda i: (i, 0)),
            pl.BlockSpec((_M, _D), lambda i: (0, 0)),
            pl.BlockSpec((_D, _M), lambda i: (0, 0)),
        ],
        out_specs=[
            pl.BlockSpec((1, _BLK), lambda i: (0, i)),
            pl.BlockSpec((8, 128), lambda i: (0, 0)),
        ],
        out_shape=[
            jax.ShapeDtypeStruct((1, _N), jnp.int32),
            jax.ShapeDtypeStruct((8, 128), jnp.float32),
        ],
        scratch_shapes=[
            pltpu.VMEM((1, _M), jnp.float32),
            pltpu.VMEM((1, 1), jnp.float32),
        ],
    )(x, embedding, embedding.T)
    e128 = jnp.pad(embedding, ((0, 0), (0, 128 - _D)))
    quant = _sc_gather(e128, idx_col)
    quantised_st = quant[:, :_D].reshape(input.shape)
    codebook_loss = stats[0, 0]
    commitment_loss = stats[1, 0]
    perplexity = stats[2, 0]
    return (quantised_st, commitment_loss, codebook_loss, perplexity)
